# Initial kernel scaffold; baseline (speedup 1.0000x reference)
import jax
import jax.numpy as jnp
from jax.experimental import pallas as pl


def kernel(node_pairs, train_hedges, labels, neighborhedges, hedgetypes, nodeEmb, W1, b1, W2, b2):
    raise NotImplementedError("write your pallas kernel here")


# speedup vs baseline: 47.6320x; 47.6320x over previous
import functools

import jax
import jax.numpy as jnp
from jax import lax
from jax.experimental import pallas as pl
from jax.experimental.pallas import tpu as pltpu
from jax.experimental.pallas import tpu_sc as plsc

N_NODES_C = 100000
N_HEDGES_C = 200000
N_TYPES_C = 16
B_C = 4096
H_C = 4
S_C = 32
D_C = 128

NC = 2
NS = 16
NW = NC * NS
BPW = B_C // NW
RPW = BPW * H_C
TP_WORDS = N_HEDGES_C // 8
TP_PAD = 25088


def _sc_body(np_hbm, th_hbm, lb_hbm, nbr_hbm, tp_hbm, emb_hbm,
             pooled_hbm, nm_hbm,
             np_v, th_v, lb_v, tp_v, ne_v, hist_v, emb_v, nm_v,
             sem_ne, sem_emb):
    wid = lax.axis_index("s") * NC + lax.axis_index("c")
    base_b = wid * BPW
    base_r = wid * RPW

    pltpu.sync_copy(np_hbm.at[pl.ds(base_r, RPW)], np_v)
    ne_copies = []
    emb_copies = []
    for c in range(4):
        idx = np_v.at[pl.ds(c * 128, 128)]
        ne_copies.append(pltpu.async_copy(
            nbr_hbm.at[idx], ne_v.at[pl.ds(c * 128, 128)], sem_ne))
        emb_copies.append(pltpu.async_copy(
            emb_hbm.at[idx], emb_v.at[pl.ds(c * 128, 128)], sem_emb))

    pltpu.sync_copy(th_hbm.at[pl.ds(base_b, BPW)], th_v)
    pltpu.sync_copy(lb_hbm.at[pl.ds(base_b, BPW)], lb_v)
    pltpu.sync_copy(tp_hbm, tp_v)

    def _zero(i, _):
        hist_v[pl.ds(i * 16, 16)] = jnp.zeros((16,), jnp.float32)
        return 0
    lax.fori_loop(0, BPW, _zero, 0)

    for cp in ne_copies:
        cp.wait()

    ones = jnp.ones((16,), jnp.float32)

    def _hist(r, _):
        b = lax.shift_right_logical(r, 2)
        thv = plsc.load_gather(th_v, [jnp.full((16,), b, jnp.int32)])
        b16 = b * 16
        for k in range(2):
            e = ne_v[r, pl.ds(k * 16, 16)]
            w = plsc.load_gather(tp_v, [lax.shift_right_logical(e, 3)])
            sh = lax.shift_left(jnp.bitwise_and(e, 7), 2)
            t = jnp.bitwise_and(lax.shift_right_logical(w, sh), 15)
            m = jnp.not_equal(e, thv)
            plsc.addupdate_scatter(hist_v, [t + b16], ones, mask=m)
        return 0
    lax.fori_loop(0, RPW, _hist, 0)

    iota16 = lax.iota(jnp.int32, 16)

    def _pooled(b, _):
        hv = hist_v[pl.ds(b * 16, 16)]
        lbl = plsc.load_gather(lb_v, [jnp.full((16,), b, jnp.int32)])
        onehot = jnp.where(iota16 == lbl, 1.0, 0.0).astype(jnp.float32)
        hist_v[pl.ds(b * 16, 16)] = hv * (1.0 / 128.0) + onehot
        return 0
    lax.fori_loop(0, BPW, _pooled, 0)
    pltpu.sync_copy(hist_v, pooled_hbm.at[pl.ds(wid * (BPW * 16), BPW * 16)])

    for cp in emb_copies:
        cp.wait()

    def _emb(b, _):
        r0 = b * 4
        for d in range(8):
            s = (emb_v[r0, pl.ds(d * 16, 16)] +
                 emb_v[r0 + 1, pl.ds(d * 16, 16)] +
                 emb_v[r0 + 2, pl.ds(d * 16, 16)] +
                 emb_v[r0 + 3, pl.ds(d * 16, 16)])
            nm_v[b, pl.ds(d * 16, 16)] = s * 0.25
        return 0
    lax.fori_loop(0, BPW, _emb, 0)
    pltpu.sync_copy(nm_v, nm_hbm.at[pl.ds(base_b, BPW)])


_sc_gather = functools.partial(
    pl.kernel,
    out_type=(
        jax.ShapeDtypeStruct((B_C * 16,), jnp.float32),
        jax.ShapeDtypeStruct((B_C, D_C), jnp.float32),
    ),
    mesh=plsc.VectorSubcoreMesh(core_axis_name="c", subcore_axis_name="s"),
    compiler_params=pltpu.CompilerParams(needs_layout_passes=False,
                                         use_tc_tiling_on_sc=False),
    scratch_types=[
        pltpu.VMEM((RPW,), jnp.int32),
        pltpu.VMEM((BPW,), jnp.int32),
        pltpu.VMEM((BPW,), jnp.int32),
        pltpu.VMEM((TP_PAD,), jnp.int32),
        pltpu.VMEM((RPW, S_C), jnp.int32),
        pltpu.VMEM((BPW * 16,), jnp.float32),
        pltpu.VMEM((RPW, D_C), jnp.float32),
        pltpu.VMEM((BPW, D_C), jnp.float32),
        pltpu.SemaphoreType.DMA,
        pltpu.SemaphoreType.DMA,
    ],
)(_sc_body)


def _tc_body(pooled_ref, nm_ref, w1_ref, b1_ref, w2_ref, b2_ref,
             sc_ref, v2_ref):
    v1 = jnp.dot(pooled_ref[...], w1_ref[...],
                 preferred_element_type=jnp.float32) + b1_ref[...]
    sc_ref[...] = jax.nn.sigmoid(v1)
    p2 = jnp.dot(nm_ref[...], w2_ref[...],
                 preferred_element_type=jnp.float32) + b2_ref[...]
    v2_ref[...] = jnp.concatenate([v1, p2], axis=1)


_tc_heads = pl.pallas_call(
    _tc_body,
    out_shape=(
        jax.ShapeDtypeStruct((B_C, N_TYPES_C), jnp.float32),
        jax.ShapeDtypeStruct((B_C, 2 * N_TYPES_C), jnp.float32),
    ),
)


@jax.jit
def kernel(node_pairs, train_hedges, labels, neighborhedges, hedgetypes,
           nodeEmb, W1, b1, W2, b2):
    np_flat = node_pairs.reshape(-1).astype(jnp.int32)
    th = train_hedges.astype(jnp.int32)
    lb = labels.astype(jnp.int32)
    nbr = neighborhedges.astype(jnp.int32)
    ht = hedgetypes.astype(jnp.int32).reshape(TP_WORDS, 8)
    shifts = (jnp.arange(8, dtype=jnp.int32) * 4)[None, :]
    tp = jnp.sum(ht << shifts, axis=1, dtype=jnp.int32)
    tp = jnp.concatenate([tp, jnp.zeros((TP_PAD - TP_WORDS,), jnp.int32)])

    pooled_flat, node_mean = _sc_gather(np_flat, th, lb, nbr, tp, nodeEmb)
    pooled = pooled_flat.reshape(B_C, N_TYPES_C)

    scores, vector2 = _tc_heads(pooled, node_mean, W1, b1.reshape(1, -1),
                                W2, b2.reshape(1, -1))
    return (scores, vector2)


# speedup vs baseline: 54.5386x; 1.1450x over previous
import functools

import jax
import jax.numpy as jnp
from jax import lax
from jax.experimental import pallas as pl
from jax.experimental.pallas import tpu as pltpu
from jax.experimental.pallas import tpu_sc as plsc

N_NODES_C = 100000
N_HEDGES_C = 200000
N_TYPES_C = 16
B_C = 4096
H_C = 4
S_C = 32
D_C = 128

NC = 2
NS = 16
NW = NC * NS
BPW = B_C // NW
RPW = BPW * H_C
NCHUNK = 4
CR = RPW // NCHUNK
TP_WORDS = N_HEDGES_C // 8
TP_PAD = 25088
NBR_ROWS = N_NODES_C * S_C // D_C


def _sc_body(np_hbm, th_hbm, lb_hbm, nbr_hbm, tp_hbm, emb_hbm,
             pooled_hbm, nm_hbm,
             np_v, gq_v, cb_v, th_v, lb_v, tp_v,
             nb0, nb1, eb0, eb1, hist_v, nm_v,
             sem_ne, sem_emb):
    wid = lax.axis_index("s") * NC + lax.axis_index("c")
    base_b = wid * BPW
    base_r = wid * RPW
    iota16 = lax.iota(jnp.int32, 16)
    ones = jnp.ones((16,), jnp.float32)

    pltpu.sync_copy(np_hbm.at[pl.ds(base_r, RPW)], np_v)

    @plsc.parallel_loop(0, RPW // 16)
    def _gq(i):
        v = np_v[pl.ds(i * 16, 16)]
        gq_v[pl.ds(i * 16, 16)] = lax.shift_right_logical(v, 2)
        cb_v[pl.ds(i * 16, 16)] = lax.shift_left(jnp.bitwise_and(v, 3), 5)

    nbufs = [nb0, nb1]
    ebufs = [eb0, eb1]
    ne_cp = [None] * NCHUNK
    emb_cp = [None] * NCHUNK

    def _fire_ne(c):
        ne_cp[c] = pltpu.async_copy(
            nbr_hbm.at[gq_v.at[pl.ds(c * CR, CR)]], nbufs[c % 2], sem_ne)

    def _fire_emb(c):
        emb_cp[c] = pltpu.async_copy(
            emb_hbm.at[np_v.at[pl.ds(c * CR, CR)]], ebufs[c % 2], sem_emb)

    _fire_ne(0)
    _fire_ne(1)
    _fire_emb(0)
    _fire_emb(1)

    pltpu.sync_copy(th_hbm.at[pl.ds(base_b, BPW)], th_v)
    pltpu.sync_copy(lb_hbm.at[pl.ds(base_b, BPW)], lb_v)
    pltpu.sync_copy(tp_hbm, tp_v)

    @plsc.parallel_loop(0, BPW)
    def _zero(i):
        hist_v[pl.ds(i * 16, 16)] = jnp.zeros((16,), jnp.float32)

    for c in range(NCHUNK):
        ne_cp[c].wait()
        nb = nbufs[c % 2]

        @plsc.parallel_loop(0, CR)
        def _hist(r, _c=c, _nb=nb):
            row = _c * CR + r
            b = lax.shift_right_logical(row, 2)
            rowv = jnp.full((16,), r, jnp.int32)
            thv = plsc.load_gather(th_v, [jnp.full((16,), b, jnp.int32)])
            cb = plsc.load_gather(cb_v, [jnp.full((16,), row, jnp.int32)])
            b16 = b * 16
            for k in range(2):
                col = cb + (iota16 + (k * 16))
                e = plsc.load_gather(_nb, [rowv, col])
                w = plsc.load_gather(tp_v, [lax.shift_right_logical(e, 3)])
                sh = lax.shift_left(jnp.bitwise_and(e, 7), 2)
                t = jnp.bitwise_and(lax.shift_right_logical(w, sh), 15)
                m = jnp.not_equal(e, thv)
                plsc.addupdate_scatter(hist_v, [t + b16], ones, mask=m)

        if c + 2 < NCHUNK:
            _fire_ne(c + 2)

    @plsc.parallel_loop(0, BPW)
    def _pooled(b):
        hv = hist_v[pl.ds(b * 16, 16)]
        lbl = plsc.load_gather(lb_v, [jnp.full((16,), b, jnp.int32)])
        onehot = jnp.where(iota16 == lbl, 1.0, 0.0).astype(jnp.float32)
        hist_v[pl.ds(b * 16, 16)] = hv * (1.0 / 128.0) + onehot

    pltpu.sync_copy(hist_v, pooled_hbm.at[pl.ds(wid * (BPW * 16), BPW * 16)])

    for c in range(NCHUNK):
        emb_cp[c].wait()
        eb = ebufs[c % 2]

        @plsc.parallel_loop(0, CR // 4)
        def _emb(i, _c=c, _eb=eb):
            b = _c * (CR // 4) + i
            r0 = i * 4
            for d in range(8):
                s = (_eb[r0, pl.ds(d * 16, 16)] +
                     _eb[r0 + 1, pl.ds(d * 16, 16)] +
                     _eb[r0 + 2, pl.ds(d * 16, 16)] +
                     _eb[r0 + 3, pl.ds(d * 16, 16)])
                nm_v[b, pl.ds(d * 16, 16)] = s * 0.25

        if c + 2 < NCHUNK:
            _fire_emb(c + 2)

    pltpu.sync_copy(nm_v, nm_hbm.at[pl.ds(base_b, BPW)])


_sc_gather = functools.partial(
    pl.kernel,
    out_type=(
        jax.ShapeDtypeStruct((B_C * 16,), jnp.float32),
        jax.ShapeDtypeStruct((B_C, D_C), jnp.float32),
    ),
    mesh=plsc.VectorSubcoreMesh(core_axis_name="c", subcore_axis_name="s"),
    compiler_params=pltpu.CompilerParams(needs_layout_passes=False,
                                         use_tc_tiling_on_sc=False),
    scratch_types=[
        pltpu.VMEM((RPW,), jnp.int32),
        pltpu.VMEM((RPW,), jnp.int32),
        pltpu.VMEM((RPW,), jnp.int32),
        pltpu.VMEM((BPW,), jnp.int32),
        pltpu.VMEM((BPW,), jnp.int32),
        pltpu.VMEM((TP_PAD,), jnp.int32),
        pltpu.VMEM((CR, D_C), jnp.int32),
        pltpu.VMEM((CR, D_C), jnp.int32),
        pltpu.VMEM((CR, D_C), jnp.float32),
        pltpu.VMEM((CR, D_C), jnp.float32),
        pltpu.VMEM((BPW * 16,), jnp.float32),
        pltpu.VMEM((BPW, D_C), jnp.float32),
        pltpu.SemaphoreType.DMA,
        pltpu.SemaphoreType.DMA,
    ],
)(_sc_body)


def _tc_body(pooled_ref, nm_ref, w1_ref, b1_ref, w2_ref, b2_ref,
             sc_ref, v2_ref):
    v1 = jnp.dot(pooled_ref[...], w1_ref[...],
                 preferred_element_type=jnp.float32) + b1_ref[...]
    sc_ref[...] = jax.nn.sigmoid(v1)
    p2 = jnp.dot(nm_ref[...], w2_ref[...],
                 preferred_element_type=jnp.float32) + b2_ref[...]
    v2_ref[...] = jnp.concatenate([v1, p2], axis=1)


_tc_heads = pl.pallas_call(
    _tc_body,
    out_shape=(
        jax.ShapeDtypeStruct((B_C, N_TYPES_C), jnp.float32),
        jax.ShapeDtypeStruct((B_C, 2 * N_TYPES_C), jnp.float32),
    ),
)


@jax.jit
def kernel(node_pairs, train_hedges, labels, neighborhedges, hedgetypes,
           nodeEmb, W1, b1, W2, b2):
    np_flat = node_pairs.reshape(-1).astype(jnp.int32)
    th = train_hedges.astype(jnp.int32)
    lb = labels.astype(jnp.int32)
    nbr = neighborhedges.astype(jnp.int32).reshape(NBR_ROWS, D_C)
    ht = hedgetypes.astype(jnp.int32).reshape(TP_WORDS, 8)
    shifts = (jnp.arange(8, dtype=jnp.int32) * 4)[None, :]
    tp = jnp.sum(ht << shifts, axis=1, dtype=jnp.int32)
    tp = jnp.concatenate([tp, jnp.zeros((TP_PAD - TP_WORDS,), jnp.int32)])

    pooled_flat, node_mean = _sc_gather(np_flat, th, lb, nbr, tp, nodeEmb)
    pooled = pooled_flat.reshape(B_C, N_TYPES_C)

    scores, vector2 = _tc_heads(pooled, node_mean, W1, b1.reshape(1, -1),
                                W2, b2.reshape(1, -1))
    return (scores, vector2)
